# baseline (speedup 1.0000x reference)
import functools

import jax
import jax.numpy as jnp
import numpy as np
from jax import lax
from jax.experimental import pallas as pl
from jax.experimental.pallas import tpu as pltpu
from jax.experimental.pallas import tpu_sc as plsc

_B, _N = 128, 106
_IN_H, _IN_W = 512, 512
_UPSCALE = 4
_STRIDE = 4
_OUT_H = int(_IN_H * _UPSCALE / _STRIDE)
_OUT_W = int(_IN_W * _UPSCALE / _STRIDE)
_SIGMA = 2.0
_RADIUS = int(_SIGMA * 3)
_KSIZE = 2 * _RADIUS + 1

_NC, _NS, _L = 2, 16, 16
_NW = _NC * _NS
_ROWS = _B * _N
_RPW = _ROWS // _NW
_JOB = 112
_JOBS = (_JOB, _JOB, _JOB, _RPW - 3 * _JOB)
_CPAD = 448

_TC_ROWS = 848


def _fill_windows(buf, cs_v, os_v, off, nrows, g16, iota16):
    ngrp = (nrows + _L - 1) // _L

    def grp_body(gi, _):
        r0 = gi * _L
        c16 = cs_v[pl.ds(off + r0, _L)].astype(jnp.int32)
        o16 = os_v[pl.ds(off + r0, _L)].astype(jnp.int32)
        ul_c, ul_o = c16 - _RADIUS, o16 - _RADIUS
        br_c, br_o = c16 + (_RADIUS + 1), o16 + (_RADIUS + 1)

        def in_img(px, py):
            return jnp.logical_not((px < 0) | (px > _OUT_W) | (py < 0) | (py > _OUT_H))

        valid = (in_img(ul_c, ul_o) | in_img(br_c, br_o)).astype(jnp.int32)
        for k in range(_L):
            ok = (valid[k] != 0) & (r0 + k < nrows)
            col = iota16 + ul_c[k]
            mask = (col >= 0) & (col < _OUT_W) & (iota16 < _KSIZE) & ok
            row_idx = jnp.full((_L,), r0 + k, jnp.int32)
            plsc.store_scatter(buf, [row_idx, col], g16, mask=mask)
        return 0

    lax.fori_loop(0, ngrp, grp_body, 0)


def _zero_buf(buf, nrows):
    z = jnp.zeros((_L,), jnp.float32)

    def row_body(r, _):
        for c in range(_OUT_W // _L):
            buf[r, pl.ds(c * _L, _L)] = z
        return 0

    lax.fori_loop(0, nrows, row_body, 0)


def _sc_gauss(xs_hbm, ys_hbm, vx_hbm, xs_v, ys_v, buf0, buf1,
              sem_x, sem_y, sem0, sem1):
    wid = lax.axis_index("s") * _NC + lax.axis_index("c")
    r_base = wid * _RPW

    cpx = pltpu.async_copy(xs_hbm.at[pl.ds(r_base, _RPW)], xs_v.at[pl.ds(0, _RPW)], sem_x)
    cpy = pltpu.async_copy(ys_hbm.at[pl.ds(r_base, _RPW)], ys_v.at[pl.ds(0, _RPW)], sem_y)

    iota16 = lax.iota(jnp.int32, _L)
    d = (iota16 - _RADIUS).astype(jnp.float32)
    g16 = jnp.exp(d * d * (-1.0 / (2.0 * _SIGMA * _SIGMA)))

    _zero_buf(buf0, _JOB)
    _zero_buf(buf1, _JOB)
    cpx.wait()
    cpy.wait()

    bufs = (buf0, buf1)
    sems = (sem0, sem1)
    pending = [None, None]
    jobs = []
    off = 0
    for nrows in _JOBS:
        jobs.append((off, nrows))
        off += nrows
    for j, (off, nrows) in enumerate(jobs):
        phase = j % 2
        buf = bufs[phase]
        if pending[phase] is not None:
            pending[phase].wait()
            _zero_buf(buf, _JOB)
        _fill_windows(buf, xs_v, ys_v, off, nrows, g16, iota16)
        cp = pltpu.async_copy(
            buf.at[pl.ds(0, nrows)], vx_hbm.at[pl.ds(r_base + off, nrows)], sems[phase]
        )
        pending[phase] = cp
    pending[0].wait()
    pending[1].wait()


def _tc_gauss(ys_ref, xs_ref, vy_ref):
    cf = jnp.trunc(ys_ref[...])
    of = jnp.trunc(xs_ref[...])
    ul_c, ul_o = cf - _RADIUS, of - _RADIUS
    br_c, br_o = cf + (_RADIUS + 1), of + (_RADIUS + 1)

    def in_img(px, py):
        return jnp.logical_not((px < 0) | (px > _OUT_W) | (py < 0) | (py > _OUT_H))

    valid = in_img(ul_c, ul_o) | in_img(br_c, br_o)
    neg_c = -float(np.log2(np.e)) / (2.0 * _SIGMA * _SIGMA)
    pen = jnp.where(valid, 0.0, -jnp.inf)
    r2 = float(_RADIUS * _RADIUS)

    wf = jax.lax.broadcasted_iota(jnp.int32, (_TC_ROWS, _OUT_H), 1).astype(jnp.float32)
    dd = wf - cf
    d2 = dd * dd
    v = jnp.exp2(d2 * neg_c + pen)
    vy_ref[...] = jnp.where(d2 <= r2, v, 0.0)


def kernel(lmks):
    scaled = lmks * (_UPSCALE / _STRIDE)
    xs = scaled[:, :, 0].T.reshape(_ROWS)
    ys = scaled[:, :, 1].T.reshape(_ROWS)

    mesh = plsc.VectorSubcoreMesh(core_axis_name="c", subcore_axis_name="s")
    sc_call = functools.partial(
        pl.kernel,
        mesh=mesh,
        out_type=jax.ShapeDtypeStruct((_ROWS, _OUT_W), jnp.float32),
        scratch_types=[
            pltpu.VMEM((_CPAD,), jnp.float32),
            pltpu.VMEM((_CPAD,), jnp.float32),
            pltpu.VMEM((_JOB, _OUT_W), jnp.float32),
            pltpu.VMEM((_JOB, _OUT_W), jnp.float32),
            pltpu.SemaphoreType.DMA,
            pltpu.SemaphoreType.DMA,
            pltpu.SemaphoreType.DMA,
            pltpu.SemaphoreType.DMA,
        ],
        compiler_params=pltpu.CompilerParams(
            needs_layout_passes=False, skip_device_barrier=True
        ),
    )(_sc_gauss)
    fx = sc_call(xs, ys)

    ys2 = ys.reshape(_ROWS, 1)
    xs2 = xs.reshape(_ROWS, 1)
    fy = pl.pallas_call(
        _tc_gauss,
        grid=(_ROWS // _TC_ROWS,),
        in_specs=[
            pl.BlockSpec((_TC_ROWS, 1), lambda i: (i, 0)),
            pl.BlockSpec((_TC_ROWS, 1), lambda i: (i, 0)),
        ],
        out_specs=pl.BlockSpec((_TC_ROWS, _OUT_H), lambda i: (i, 0)),
        out_shape=jax.ShapeDtypeStruct((_ROWS, _OUT_H), jnp.float32),
    )(ys2, xs2)

    vx = fx.reshape(_N, _B, _OUT_W).transpose(1, 0, 2)
    vy = fy.reshape(_N, _B, _OUT_H).transpose(1, 0, 2)
    return vx, vy


# speedup vs baseline: 1.2836x; 1.2836x over previous
import functools

import jax
import jax.numpy as jnp
import numpy as np
from jax import lax
from jax.experimental import pallas as pl
from jax.experimental.pallas import tpu as pltpu
from jax.experimental.pallas import tpu_sc as plsc

_B, _N = 128, 106
_IN_H, _IN_W = 512, 512
_UPSCALE = 4
_STRIDE = 4
_OUT_H = int(_IN_H * _UPSCALE / _STRIDE)
_OUT_W = int(_IN_W * _UPSCALE / _STRIDE)
_SIGMA = 2.0
_RADIUS = int(_SIGMA * 3)
_KSIZE = 2 * _RADIUS + 1

_NC, _NS, _L = 2, 16, 16
_NW = _NC * _NS
_ROWS = _B * _N
_RPW = _ROWS // _NW
_JOB = 112
_JOBS = (_JOB, _JOB, _JOB, _RPW - 3 * _JOB)
_CPAD = 448

_TC_ROWS = 848


def _fill_windows(buf, cs_v, os_v, off, nrows, g16, iota16):
    ngrp = (nrows + _L - 1) // _L

    def grp_body(gi, _):
        r0 = gi * _L
        c16 = cs_v[pl.ds(off + r0, _L)].astype(jnp.int32)
        o16 = os_v[pl.ds(off + r0, _L)].astype(jnp.int32)
        ul_c, ul_o = c16 - _RADIUS, o16 - _RADIUS
        br_c, br_o = c16 + (_RADIUS + 1), o16 + (_RADIUS + 1)

        def in_img(px, py):
            return jnp.logical_not((px < 0) | (px > _OUT_W) | (py < 0) | (py > _OUT_H))

        valid = (in_img(ul_c, ul_o) | in_img(br_c, br_o)).astype(jnp.int32)
        for k in range(_L):
            ok = (valid[k] != 0) & (r0 + k < nrows)
            col = iota16 + ul_c[k]
            mask = (col >= 0) & (col < _OUT_W) & (iota16 < _KSIZE) & ok
            row_idx = jnp.full((_L,), r0 + k, jnp.int32)
            plsc.store_scatter(buf, [row_idx, col], g16, mask=mask)
        return 0

    lax.fori_loop(0, ngrp, grp_body, 0)


def _zero_buf(buf, nrows):
    z = jnp.zeros((_L,), jnp.float32)

    def row_body(r, _):
        for c in range(_OUT_W // _L):
            buf[r, pl.ds(c * _L, _L)] = z
        return 0

    lax.fori_loop(0, nrows, row_body, 0)


def _sc_gauss(xs_hbm, ys_hbm, vx_hbm, xs_v, ys_v, buf0, buf1,
              sem_x, sem_y, sem0, sem1):
    wid = lax.axis_index("s") * _NC + lax.axis_index("c")
    r_base = wid * _RPW

    cpx = pltpu.async_copy(xs_hbm.at[pl.ds(r_base, _RPW)], xs_v.at[pl.ds(0, _RPW)], sem_x)
    cpy = pltpu.async_copy(ys_hbm.at[pl.ds(r_base, _RPW)], ys_v.at[pl.ds(0, _RPW)], sem_y)

    iota16 = lax.iota(jnp.int32, _L)
    d = (iota16 - _RADIUS).astype(jnp.float32)
    g16 = jnp.exp(d * d * (-1.0 / (2.0 * _SIGMA * _SIGMA)))

    _zero_buf(buf0, _JOB)
    _zero_buf(buf1, _JOB)
    cpx.wait()
    cpy.wait()

    bufs = (buf0, buf1)
    sems = (sem0, sem1)
    pending = [None, None]
    jobs = []
    off = 0
    for nrows in _JOBS:
        jobs.append((off, nrows))
        off += nrows
    for j, (off, nrows) in enumerate(jobs):
        phase = j % 2
        buf = bufs[phase]
        if pending[phase] is not None:
            pending[phase].wait()
            _zero_buf(buf, _JOB)
        _fill_windows(buf, xs_v, ys_v, off, nrows, g16, iota16)
        cp = pltpu.async_copy(
            buf.at[pl.ds(0, nrows)], vx_hbm.at[pl.ds(r_base + off, nrows)], sems[phase]
        )
        pending[phase] = cp
    pending[0].wait()
    pending[1].wait()


def _tc_gauss(ys_ref, xs_ref, vy_ref):
    nblk = _ROWS // _TC_ROWS
    jd = pl.program_id(0)
    onehot = (jax.lax.broadcasted_iota(jnp.int32, (_TC_ROWS, nblk), 1) == jd)
    onehot = onehot.astype(jnp.float32)
    cf = jnp.trunc(jnp.sum(ys_ref[...] * onehot, axis=1, keepdims=True))
    of = jnp.trunc(jnp.sum(xs_ref[...] * onehot, axis=1, keepdims=True))
    ul_c, ul_o = cf - _RADIUS, of - _RADIUS
    br_c, br_o = cf + (_RADIUS + 1), of + (_RADIUS + 1)

    def in_img(px, py):
        return jnp.logical_not((px < 0) | (px > _OUT_W) | (py < 0) | (py > _OUT_H))

    valid = in_img(ul_c, ul_o) | in_img(br_c, br_o)
    neg_c = -float(np.log2(np.e)) / (2.0 * _SIGMA * _SIGMA)
    pen = jnp.where(valid, 0.0, -jnp.inf)
    r2 = float(_RADIUS * _RADIUS)

    wf = jax.lax.broadcasted_iota(jnp.int32, (_TC_ROWS, _OUT_H), 1).astype(jnp.float32)
    dd = wf - cf
    d2 = dd * dd
    v = jnp.exp2(d2 * neg_c + pen)
    vy_ref[...] = jnp.where(d2 <= r2, v, 0.0)


def kernel(lmks):
    scaled = lmks * (_UPSCALE / _STRIDE)
    xs = scaled[:, :, 0].T.reshape(_ROWS)
    ys = scaled[:, :, 1].T.reshape(_ROWS)

    mesh = plsc.VectorSubcoreMesh(core_axis_name="c", subcore_axis_name="s")
    sc_call = functools.partial(
        pl.kernel,
        mesh=mesh,
        out_type=jax.ShapeDtypeStruct((_ROWS, _OUT_W), jnp.float32),
        scratch_types=[
            pltpu.VMEM((_CPAD,), jnp.float32),
            pltpu.VMEM((_CPAD,), jnp.float32),
            pltpu.VMEM((_JOB, _OUT_W), jnp.float32),
            pltpu.VMEM((_JOB, _OUT_W), jnp.float32),
            pltpu.SemaphoreType.DMA,
            pltpu.SemaphoreType.DMA,
            pltpu.SemaphoreType.DMA,
            pltpu.SemaphoreType.DMA,
        ],
        compiler_params=pltpu.CompilerParams(
            needs_layout_passes=False, skip_device_barrier=True
        ),
    )(_sc_gauss)
    fx = sc_call(xs, ys)

    nblk = _ROWS // _TC_ROWS
    ys2 = ys.reshape(nblk, _TC_ROWS).T
    xs2 = xs.reshape(nblk, _TC_ROWS).T
    fy = pl.pallas_call(
        _tc_gauss,
        grid=(nblk,),
        in_specs=[
            pl.BlockSpec((_TC_ROWS, nblk), lambda i: (0, 0)),
            pl.BlockSpec((_TC_ROWS, nblk), lambda i: (0, 0)),
        ],
        out_specs=pl.BlockSpec((_TC_ROWS, _OUT_H), lambda i: (i, 0)),
        out_shape=jax.ShapeDtypeStruct((_ROWS, _OUT_H), jnp.float32),
    )(ys2, xs2)

    vx = fx.reshape(_N, _B, _OUT_W).transpose(1, 0, 2)
    vy = fy.reshape(_N, _B, _OUT_H).transpose(1, 0, 2)
    return vx, vy


# speedup vs baseline: 1.2984x; 1.0115x over previous
import functools

import jax
import jax.numpy as jnp
import numpy as np
from jax import lax
from jax.experimental import pallas as pl
from jax.experimental.pallas import tpu as pltpu
from jax.experimental.pallas import tpu_sc as plsc

_B, _N = 128, 106
_IN_H, _IN_W = 512, 512
_UPSCALE = 4
_STRIDE = 4
_OUT_H = int(_IN_H * _UPSCALE / _STRIDE)
_OUT_W = int(_IN_W * _UPSCALE / _STRIDE)
_SIGMA = 2.0
_RADIUS = int(_SIGMA * 3)
_KSIZE = 2 * _RADIUS + 1

_NC, _NS, _L = 2, 16, 16
_NW = _NC * _NS
_ROWS = _B * _N
_RPW = _ROWS // _NW
_JOB = 112
_JOBS = (_JOB, _JOB, _JOB, _RPW - 3 * _JOB)
_CPAD = 448

_TC_ROWS = 1696


def _fill_windows(buf, cs_v, os_v, off, nrows, g16, iota16):
    ngrp = (nrows + _L - 1) // _L

    def grp_body(gi, _):
        r0 = gi * _L
        c16 = cs_v[pl.ds(off + r0, _L)].astype(jnp.int32)
        o16 = os_v[pl.ds(off + r0, _L)].astype(jnp.int32)
        ul_c, ul_o = c16 - _RADIUS, o16 - _RADIUS
        br_c, br_o = c16 + (_RADIUS + 1), o16 + (_RADIUS + 1)

        def in_img(px, py):
            return jnp.logical_not((px < 0) | (px > _OUT_W) | (py < 0) | (py > _OUT_H))

        valid = (in_img(ul_c, ul_o) | in_img(br_c, br_o)).astype(jnp.int32)
        for k in range(_L):
            ok = (valid[k] != 0) & (r0 + k < nrows)
            col = iota16 + ul_c[k]
            mask = (col >= 0) & (col < _OUT_W) & (iota16 < _KSIZE) & ok
            row_idx = jnp.full((_L,), r0 + k, jnp.int32)
            plsc.store_scatter(buf, [row_idx, col], g16, mask=mask)
        return 0

    lax.fori_loop(0, ngrp, grp_body, 0)


def _zero_buf(buf, nrows):
    z = jnp.zeros((_L,), jnp.float32)

    def row_body(r, _):
        for c in range(_OUT_W // _L):
            buf[r, pl.ds(c * _L, _L)] = z
        return 0

    lax.fori_loop(0, nrows, row_body, 0)


def _sc_gauss(xs_hbm, ys_hbm, vx_hbm, xs_v, ys_v, buf0, buf1,
              sem_x, sem_y, sem0, sem1):
    wid = lax.axis_index("s") * _NC + lax.axis_index("c")
    r_base = wid * _RPW

    cpx = pltpu.async_copy(xs_hbm.at[pl.ds(r_base, _RPW)], xs_v.at[pl.ds(0, _RPW)], sem_x)
    cpy = pltpu.async_copy(ys_hbm.at[pl.ds(r_base, _RPW)], ys_v.at[pl.ds(0, _RPW)], sem_y)

    iota16 = lax.iota(jnp.int32, _L)
    d = (iota16 - _RADIUS).astype(jnp.float32)
    g16 = jnp.exp(d * d * (-1.0 / (2.0 * _SIGMA * _SIGMA)))

    _zero_buf(buf0, _JOB)
    _zero_buf(buf1, _JOB)
    cpx.wait()
    cpy.wait()

    bufs = (buf0, buf1)
    sems = (sem0, sem1)
    pending = [None, None]
    jobs = []
    off = 0
    for nrows in _JOBS:
        jobs.append((off, nrows))
        off += nrows
    for j, (off, nrows) in enumerate(jobs):
        phase = j % 2
        buf = bufs[phase]
        if pending[phase] is not None:
            pending[phase].wait()
            _zero_buf(buf, _JOB)
        _fill_windows(buf, xs_v, ys_v, off, nrows, g16, iota16)
        cp = pltpu.async_copy(
            buf.at[pl.ds(0, nrows)], vx_hbm.at[pl.ds(r_base + off, nrows)], sems[phase]
        )
        pending[phase] = cp
    pending[0].wait()
    pending[1].wait()


def _tc_gauss(ys_ref, xs_ref, vy_ref):
    nblk = _ROWS // _TC_ROWS
    jd = pl.program_id(0)
    onehot = (jax.lax.broadcasted_iota(jnp.int32, (_TC_ROWS, nblk), 1) == jd)
    onehot = onehot.astype(jnp.float32)
    cf = jnp.trunc(jnp.sum(ys_ref[...] * onehot, axis=1, keepdims=True))
    of = jnp.trunc(jnp.sum(xs_ref[...] * onehot, axis=1, keepdims=True))
    ul_c, ul_o = cf - _RADIUS, of - _RADIUS
    br_c, br_o = cf + (_RADIUS + 1), of + (_RADIUS + 1)

    def in_img(px, py):
        return jnp.logical_not((px < 0) | (px > _OUT_W) | (py < 0) | (py > _OUT_H))

    valid = in_img(ul_c, ul_o) | in_img(br_c, br_o)
    neg_c = -float(np.log2(np.e)) / (2.0 * _SIGMA * _SIGMA)
    pen = jnp.where(valid, 0.0, -jnp.inf)
    r2 = float(_RADIUS * _RADIUS)

    wf = jax.lax.broadcasted_iota(jnp.int32, (_TC_ROWS, _OUT_H), 1).astype(jnp.float32)
    dd = wf - cf
    d2 = dd * dd
    v = jnp.exp2(d2 * neg_c + pen)
    vy_ref[...] = jnp.where(d2 <= r2, v, 0.0)


def kernel(lmks):
    scaled = lmks * (_UPSCALE / _STRIDE)
    xs = scaled[:, :, 0].T.reshape(_ROWS)
    ys = scaled[:, :, 1].T.reshape(_ROWS)

    mesh = plsc.VectorSubcoreMesh(core_axis_name="c", subcore_axis_name="s")
    sc_call = functools.partial(
        pl.kernel,
        mesh=mesh,
        out_type=jax.ShapeDtypeStruct((_ROWS, _OUT_W), jnp.float32),
        scratch_types=[
            pltpu.VMEM((_CPAD,), jnp.float32),
            pltpu.VMEM((_CPAD,), jnp.float32),
            pltpu.VMEM((_JOB, _OUT_W), jnp.float32),
            pltpu.VMEM((_JOB, _OUT_W), jnp.float32),
            pltpu.SemaphoreType.DMA,
            pltpu.SemaphoreType.DMA,
            pltpu.SemaphoreType.DMA,
            pltpu.SemaphoreType.DMA,
        ],
        compiler_params=pltpu.CompilerParams(
            needs_layout_passes=False, skip_device_barrier=True
        ),
    )(_sc_gauss)
    fx = sc_call(xs, ys)

    nblk = _ROWS // _TC_ROWS
    ys2 = ys.reshape(nblk, _TC_ROWS).T
    xs2 = xs.reshape(nblk, _TC_ROWS).T
    fy = pl.pallas_call(
        _tc_gauss,
        grid=(nblk,),
        in_specs=[
            pl.BlockSpec((_TC_ROWS, nblk), lambda i: (0, 0)),
            pl.BlockSpec((_TC_ROWS, nblk), lambda i: (0, 0)),
        ],
        out_specs=pl.BlockSpec((_TC_ROWS, _OUT_H), lambda i: (i, 0)),
        out_shape=jax.ShapeDtypeStruct((_ROWS, _OUT_H), jnp.float32),
    )(ys2, xs2)

    vx = fx.reshape(_N, _B, _OUT_W).transpose(1, 0, 2)
    vy = fy.reshape(_N, _B, _OUT_H).transpose(1, 0, 2)
    return vx, vy


# speedup vs baseline: 1.3431x; 1.0344x over previous
import functools

import jax
import jax.numpy as jnp
import numpy as np
from jax import lax
from jax.experimental import pallas as pl
from jax.experimental.pallas import tpu as pltpu
from jax.experimental.pallas import tpu_sc as plsc

_B, _N = 128, 106
_IN_H, _IN_W = 512, 512
_UPSCALE = 4
_STRIDE = 4
_OUT_H = int(_IN_H * _UPSCALE / _STRIDE)
_OUT_W = int(_IN_W * _UPSCALE / _STRIDE)
_SIGMA = 2.0
_RADIUS = int(_SIGMA * 3)
_KSIZE = 2 * _RADIUS + 1

_NC, _NS, _L = 2, 16, 16
_NW = _NC * _NS
_ROWS = _B * _N
_RPW = _ROWS // _NW
_JOB = 112
_JOBS = (_JOB, _JOB, _JOB, _RPW - 3 * _JOB)
_CPAD = 448

_TC_ROWS = 1696


def _fill_windows(buf, cs_v, os_v, off, nrows, g16, iota16):
    ngrp = (nrows + _L - 1) // _L

    def grp_body(gi, _):
        r0 = gi * _L
        c16 = cs_v[pl.ds(off + r0, _L)].astype(jnp.int32)
        o16 = os_v[pl.ds(off + r0, _L)].astype(jnp.int32)
        ul_c, ul_o = c16 - _RADIUS, o16 - _RADIUS
        br_c, br_o = c16 + (_RADIUS + 1), o16 + (_RADIUS + 1)

        def in_img(px, py):
            return jnp.logical_not((px < 0) | (px > _OUT_W) | (py < 0) | (py > _OUT_H))

        valid = (in_img(ul_c, ul_o) | in_img(br_c, br_o)).astype(jnp.int32)
        for k in range(_L):
            ok = (valid[k] != 0) & (r0 + k < nrows)
            col = iota16 + ul_c[k]
            mask = (col >= 0) & (col < _OUT_W) & (iota16 < _KSIZE) & ok
            row_idx = jnp.full((_L,), r0 + k, jnp.int32)
            plsc.store_scatter(buf, [row_idx, col], g16, mask=mask)
        return 0

    lax.fori_loop(0, ngrp, grp_body, 0)


def _zero_buf(buf, nrows):
    z = jnp.zeros((_L,), jnp.float32)

    def row_body(r, _):
        for c in range(_OUT_W // _L):
            buf[r, pl.ds(c * _L, _L)] = z
        return 0

    lax.fori_loop(0, nrows, row_body, 0)


def _sc_gauss(xs_hbm, ys_hbm, vx_hbm, xs_v, ys_v, buf0, buf1,
              sem_x, sem_y, sem0, sem1):
    wid = lax.axis_index("s") * _NC + lax.axis_index("c")
    r_base = wid * _RPW

    cpx = pltpu.async_copy(xs_hbm.at[pl.ds(r_base, _RPW)], xs_v.at[pl.ds(0, _RPW)], sem_x)
    cpy = pltpu.async_copy(ys_hbm.at[pl.ds(r_base, _RPW)], ys_v.at[pl.ds(0, _RPW)], sem_y)

    iota16 = lax.iota(jnp.int32, _L)
    d = (iota16 - _RADIUS).astype(jnp.float32)
    g16 = jnp.exp(d * d * (-1.0 / (2.0 * _SIGMA * _SIGMA)))

    _zero_buf(buf0, _JOB)
    _zero_buf(buf1, _JOB)
    cpx.wait()
    cpy.wait()

    bufs = (buf0, buf1)
    sems = (sem0, sem1)
    pending = [None, None]
    jobs = []
    off = 0
    for nrows in _JOBS:
        jobs.append((off, nrows))
        off += nrows
    for j, (off, nrows) in enumerate(jobs):
        phase = j % 2
        buf = bufs[phase]
        if pending[phase] is not None:
            pending[phase].wait()
            _zero_buf(buf, _JOB)
        _fill_windows(buf, xs_v, ys_v, off, nrows, g16, iota16)
        cp = pltpu.async_copy(
            buf.at[pl.ds(0, nrows)], vx_hbm.at[pl.ds(r_base + off, nrows)], sems[phase]
        )
        pending[phase] = cp
    pending[0].wait()
    pending[1].wait()


def _tc_gauss(ys_ref, xs_ref, vy_ref):
    nblk = _ROWS // _TC_ROWS
    jd = pl.program_id(0)
    onehot = (jax.lax.broadcasted_iota(jnp.int32, (_TC_ROWS, nblk), 1) == jd)
    onehot = onehot.astype(jnp.float32)
    cf = jnp.trunc(jnp.sum(ys_ref[...] * onehot, axis=1, keepdims=True))
    of = jnp.trunc(jnp.sum(xs_ref[...] * onehot, axis=1, keepdims=True))
    ul_c, ul_o = cf - _RADIUS, of - _RADIUS
    br_c, br_o = cf + (_RADIUS + 1), of + (_RADIUS + 1)

    def in_img(px, py):
        return jnp.logical_not((px < 0) | (px > _OUT_W) | (py < 0) | (py > _OUT_H))

    valid = in_img(ul_c, ul_o) | in_img(br_c, br_o)
    neg_c = -float(np.log2(np.e)) / (2.0 * _SIGMA * _SIGMA)
    pen = jnp.where(valid, 0.0, -jnp.inf)
    r2 = float(_RADIUS * _RADIUS)

    wf = jax.lax.broadcasted_iota(jnp.int32, (_TC_ROWS, _OUT_H), 1).astype(jnp.float32)
    dd = wf - cf
    d2 = dd * dd
    v = jnp.exp2(d2 * neg_c + pen)
    vy_ref[...] = jnp.where(d2 <= r2, v, 0.0)


def kernel(lmks):
    scaled = lmks * (_UPSCALE / _STRIDE)
    xs = scaled[:, :, 0].T.reshape(_ROWS)
    ys = scaled[:, :, 1].T.reshape(_ROWS)

    mesh = plsc.VectorSubcoreMesh(core_axis_name="c", subcore_axis_name="s")
    sc_call = functools.partial(
        pl.kernel,
        mesh=mesh,
        out_type=jax.ShapeDtypeStruct((_ROWS, _OUT_W), jnp.float32),
        scratch_types=[
            pltpu.VMEM((_CPAD,), jnp.float32),
            pltpu.VMEM((_CPAD,), jnp.float32),
            pltpu.VMEM((_JOB, _OUT_W), jnp.float32),
            pltpu.VMEM((_JOB, _OUT_W), jnp.float32),
            pltpu.SemaphoreType.DMA,
            pltpu.SemaphoreType.DMA,
            pltpu.SemaphoreType.DMA,
            pltpu.SemaphoreType.DMA,
        ],
        compiler_params=pltpu.CompilerParams(
            needs_layout_passes=False, skip_device_barrier=True
        ),
    )(_sc_gauss)
    fx = sc_call(xs, ys)

    nblk = _ROWS // _TC_ROWS
    ys2 = ys.reshape(nblk, _TC_ROWS).T
    xs2 = xs.reshape(nblk, _TC_ROWS).T
    fy = pl.pallas_call(
        _tc_gauss,
        grid=(nblk,),
        in_specs=[
            pl.BlockSpec((_TC_ROWS, nblk), lambda i: (0, 0)),
            pl.BlockSpec((_TC_ROWS, nblk), lambda i: (0, 0)),
        ],
        out_specs=pl.BlockSpec((_TC_ROWS, _OUT_H), lambda i: (i, 0)),
        out_shape=jax.ShapeDtypeStruct((_ROWS, _OUT_H), jnp.float32),
        compiler_params=pltpu.CompilerParams(allow_input_fusion=[True, True]),
    )(ys2, xs2)

    vx = fx.reshape(_N, _B, _OUT_W).transpose(1, 0, 2)
    vy = fy.reshape(_N, _B, _OUT_H).transpose(1, 0, 2)
    return vx, vy
